# baseline (speedup 1.0000x reference)
import functools

import jax
import jax.numpy as jnp
from jax import lax
from jax.experimental import pallas as pl
from jax.experimental.pallas import tpu as pltpu
from jax.experimental.pallas import tpu_sc as plsc

N = 10000
E = 320000
D = 128
G = 16

NC = 2
NS = 16
NW = NC * NS
NP = 10240
RPT = NP // NS
CHUNK = 64
NCHAIN = 5
E2 = 327680
EPW = E2 // NW
NFULL = EPW // CHUNK


def _sc_agg_body(x_hbm, src_hbm, dst_hbm, out_hbm,
                 sidx0, didx0, sidx1, didx1, sidx2, didx2, sidx3, didx3,
                 sidx4, didx4, rows0, rows1, rows2, rows3, rows4, agg_s,
                 sem0, sem1, sem2, sem3, sem4):
    c = lax.axis_index("c")
    s = lax.axis_index("s")
    wid = c * NS + s
    base = wid * EPW
    sidx = (sidx0, sidx1, sidx2, sidx3, sidx4)
    didx = (didx0, didx1, didx2, didx3, didx4)
    rows = (rows0, rows1, rows2, rows3, rows4)
    sem = (sem0, sem1, sem2, sem3, sem4)

    def _start_idx(ci, sv, dv, sm):
        off = jnp.minimum(base + ci * CHUNK, E2 - CHUNK)
        pltpu.async_copy(src_hbm.at[pl.ds(off, CHUNK)], sv, sm)
        pltpu.async_copy(dst_hbm.at[pl.ds(off, CHUNK)], dv, sm)

    def _wait_idx(sv, dv, sm):
        pltpu.make_async_copy(src_hbm.at[pl.ds(0, CHUNK)], sv, sm).wait()
        pltpu.make_async_copy(dst_hbm.at[pl.ds(0, CHUNK)], dv, sm).wait()

    for b in range(NCHAIN):
        _start_idx(b, sidx[b], didx[b], sem[b])

    zeros16 = jnp.zeros((16,), jnp.float32)

    def _zrow(i, carry):
        for j in range(D // 16):
            rows0[i, pl.ds(j * 16, 16)] = zeros16
        return carry

    lax.fori_loop(0, CHUNK, _zrow, 0)
    for k in range(RPT // CHUNK):
        pltpu.sync_copy(rows0, agg_s.at[pl.ds(s * RPT + k * CHUNK, CHUNK)])
    plsc.subcore_barrier()

    def _group(j, carry):
        cb = NCHAIN * j
        for b in range(NCHAIN):
            _wait_idx(sidx[b], didx[b], sem[b])
            pltpu.async_copy(x_hbm.at[sidx[b]], rows[b], sem[b])
        for b in range(NCHAIN):
            pltpu.make_async_copy(x_hbm.at[sidx[b]], rows[b], sem[b]).wait()
            pltpu.async_copy(rows[b], agg_s.at[didx[b]], sem[b], add=True)
        for b in range(NCHAIN):
            pltpu.make_async_copy(rows[b], agg_s.at[didx[b]], sem[b]).wait()
            _start_idx(cb + NCHAIN + b, sidx[b], didx[b], sem[b])
        return carry

    lax.fori_loop(0, NFULL // NCHAIN, _group, 0)
    for b in range(NCHAIN):
        _wait_idx(sidx[b], didx[b], sem[b])

    plsc.subcore_barrier()
    pltpu.sync_copy(agg_s.at[pl.ds(s * RPT, RPT)],
                    out_hbm.at[c, pl.ds(s * RPT, RPT)])


def _sc_agg(x, src, dst):
    return pl.kernel(
        _sc_agg_body,
        out_type=jax.ShapeDtypeStruct((2, NP, D), jnp.float32),
        mesh=plsc.VectorSubcoreMesh(core_axis_name="c", subcore_axis_name="s",
                                    num_cores=NC, num_subcores=NS),
        scratch_types=(
            [pltpu.VMEM((CHUNK,), jnp.int32)] * (2 * NCHAIN)
            + [pltpu.VMEM((CHUNK, D), jnp.float32)] * NCHAIN
            + [pltpu.VMEM_SHARED((NP, D), jnp.float32)]
            + [pltpu.SemaphoreType.DMA] * NCHAIN
        ),
    )(x, src, dst)


BLK = 2000
NBLK = N // BLK


def _mlp1_body(x_ref, a0_ref, a1_ref, wa_ref, ba_ref, wb_ref, bb_ref, o_ref):
    h = x_ref[...] + a0_ref[0] + a1_ref[0]
    t = jnp.dot(h, wa_ref[...], preferred_element_type=jnp.float32) + ba_ref[...]
    t = jnp.maximum(t, 0.0)
    o = jnp.dot(t, wb_ref[...], preferred_element_type=jnp.float32) + bb_ref[...]
    o_ref[...] = jnp.maximum(o, 0.0)


def _mlp1(x, p, wa, ba, wb, bb):
    return pl.pallas_call(
        _mlp1_body,
        grid=(NBLK,),
        in_specs=[
            pl.BlockSpec((BLK, D), lambda i: (i, 0)),
            pl.BlockSpec((1, BLK, D), lambda i: (0, i, 0)),
            pl.BlockSpec((1, BLK, D), lambda i: (1, i, 0)),
            pl.BlockSpec((D, D), lambda i: (0, 0)),
            pl.BlockSpec((1, D), lambda i: (0, 0)),
            pl.BlockSpec((D, D), lambda i: (0, 0)),
            pl.BlockSpec((1, D), lambda i: (0, 0)),
        ],
        out_specs=pl.BlockSpec((BLK, D), lambda i: (i, 0)),
        out_shape=jax.ShapeDtypeStruct((N, D), jnp.float32),
    )(x, p, p, wa, ba, wb, bb)


def _mlp2_pool_body(h1_ref, a0_ref, a1_ref, wa_ref, ba_ref, wb_ref, bb_ref,
                    batch_ref, wfc_ref, bfc_ref, o_ref, acc_ref):
    i = pl.program_id(0)
    h = h1_ref[...] + a0_ref[0] + a1_ref[0]
    t = jnp.dot(h, wa_ref[...], preferred_element_type=jnp.float32) + ba_ref[...]
    t = jnp.maximum(t, 0.0)
    h2 = jnp.dot(t, wb_ref[...], preferred_element_type=jnp.float32) + bb_ref[...]
    h2 = jnp.maximum(h2, 0.0)

    onehot = (batch_ref[...] ==
              lax.broadcasted_iota(jnp.int32, (BLK, G), 1)).astype(jnp.float32)
    pooled = lax.dot_general(onehot, h2, (((0,), (0,)), ((), ())),
                             preferred_element_type=jnp.float32)

    @pl.when(i == 0)
    def _():
        acc_ref[...] = jnp.zeros_like(acc_ref)

    acc_ref[...] += pooled

    @pl.when(i == NBLK - 1)
    def _():
        logits = (jnp.dot(acc_ref[...], wfc_ref[...],
                          preferred_element_type=jnp.float32) + bfc_ref[...])
        m = jnp.max(logits, axis=1, keepdims=True)
        lse = jnp.log(jnp.sum(jnp.exp(logits - m), axis=1, keepdims=True)) + m
        o_ref[...] = logits - lse


def _mlp2_pool(h1, p, wa, ba, wb, bb, batch2d, wfc, bfc):
    return pl.pallas_call(
        _mlp2_pool_body,
        grid=(NBLK,),
        in_specs=[
            pl.BlockSpec((BLK, D), lambda i: (i, 0)),
            pl.BlockSpec((1, BLK, D), lambda i: (0, i, 0)),
            pl.BlockSpec((1, BLK, D), lambda i: (1, i, 0)),
            pl.BlockSpec((D, D), lambda i: (0, 0)),
            pl.BlockSpec((1, D), lambda i: (0, 0)),
            pl.BlockSpec((D, D), lambda i: (0, 0)),
            pl.BlockSpec((1, D), lambda i: (0, 0)),
            pl.BlockSpec((BLK, 1), lambda i: (i, 0)),
            pl.BlockSpec((D, D), lambda i: (0, 0)),
            pl.BlockSpec((1, D), lambda i: (0, 0)),
        ],
        out_specs=pl.BlockSpec((G, D), lambda i: (0, 0)),
        out_shape=jax.ShapeDtypeStruct((G, D), jnp.float32),
        scratch_shapes=[pltpu.VMEM((G, D), jnp.float32)],
    )(h1, p, p, wa, ba, wb, bb, batch2d, wfc, bfc)


def kernel(x, edge_index, batch, W1a, b1a, W1b, b1b, W2a, b2a, W2b, b2b,
           Wfc, bfc):
    src = edge_index[0].astype(jnp.int32)
    dst = edge_index[1].astype(jnp.int32)
    pad = E2 - E
    padi = jnp.arange(pad, dtype=jnp.int32)
    srcp = jnp.concatenate([src, padi % N])
    dstp = jnp.concatenate([dst, N + padi % (NP - N)])
    batch2d = batch.astype(jnp.int32).reshape(N, 1)

    p1 = _sc_agg(x, srcp, dstp)
    h1 = _mlp1(x, p1, W1a, b1a.reshape(1, D), W1b, b1b.reshape(1, D))
    p2 = _sc_agg(h1, srcp, dstp)
    return _mlp2_pool(h1, p2, W2a, b2a.reshape(1, D), W2b, b2b.reshape(1, D),
                      batch2d, Wfc, bfc.reshape(1, D))


# speedup vs baseline: 1.0026x; 1.0026x over previous
import jax
import jax.numpy as jnp
from jax import lax
from jax.experimental import pallas as pl
from jax.experimental.pallas import tpu as pltpu
from jax.experimental.pallas import tpu_sc as plsc

N = 10000
E = 320000
D = 128
G = 16

NC = 2
NS = 16
NW = NC * NS
NP = 10240
RPT = NP // NS
CHUNK = 64
NCHAIN = 5
E2 = 327680
EPW = E2 // NW
NFULL = EPW // CHUNK


def _sc_agg_body(x_hbm, src_hbm, dst_hbm, out_hbm,
                 sidx0, didx0, sidx1, didx1, sidx2, didx2, sidx3, didx3,
                 sidx4, didx4, rows0, rows1, rows2, rows3, rows4, agg_s,
                 sem0, sem1, sem2, sem3, sem4):
    c = lax.axis_index("c")
    s = lax.axis_index("s")
    wid = c * NS + s
    base = wid * EPW
    sidx = (sidx0, sidx1, sidx2, sidx3, sidx4)
    didx = (didx0, didx1, didx2, didx3, didx4)
    rows = (rows0, rows1, rows2, rows3, rows4)
    sem = (sem0, sem1, sem2, sem3, sem4)

    def _start_idx(ci, sv, dv, sm):
        off = jnp.minimum(base + ci * CHUNK, E2 - CHUNK)
        pltpu.async_copy(src_hbm.at[pl.ds(off, CHUNK)], sv, sm)
        pltpu.async_copy(dst_hbm.at[pl.ds(off, CHUNK)], dv, sm)

    def _wait_idx(sv, dv, sm):
        pltpu.make_async_copy(src_hbm.at[pl.ds(0, CHUNK)], sv, sm).wait()
        pltpu.make_async_copy(dst_hbm.at[pl.ds(0, CHUNK)], dv, sm).wait()

    for b in range(NCHAIN):
        _start_idx(b, sidx[b], didx[b], sem[b])

    zeros16 = jnp.zeros((16,), jnp.float32)

    def _zrow(i, carry):
        for j in range(D // 16):
            rows0[i, pl.ds(j * 16, 16)] = zeros16
        return carry

    lax.fori_loop(0, CHUNK, _zrow, 0)
    for k in range(RPT // CHUNK):
        pltpu.sync_copy(rows0, agg_s.at[pl.ds(s * RPT + k * CHUNK, CHUNK)])
    plsc.subcore_barrier()

    def _group(j, carry):
        cb = NCHAIN * j
        for b in range(NCHAIN):
            _wait_idx(sidx[b], didx[b], sem[b])
            pltpu.async_copy(x_hbm.at[sidx[b]], rows[b], sem[b])
        for b in range(NCHAIN):
            pltpu.make_async_copy(x_hbm.at[sidx[b]], rows[b], sem[b]).wait()
            pltpu.async_copy(rows[b], agg_s.at[didx[b]], sem[b], add=True)
        for b in range(NCHAIN):
            pltpu.make_async_copy(rows[b], agg_s.at[didx[b]], sem[b]).wait()
            _start_idx(cb + NCHAIN + b, sidx[b], didx[b], sem[b])
        return carry

    lax.fori_loop(0, NFULL // NCHAIN, _group, 0)
    for b in range(NCHAIN):
        _wait_idx(sidx[b], didx[b], sem[b])

    plsc.subcore_barrier()
    pltpu.sync_copy(agg_s.at[pl.ds(s * RPT, RPT)],
                    out_hbm.at[c, pl.ds(s * RPT, RPT)])


def _sc_agg(x, src, dst):
    return pl.kernel(
        _sc_agg_body,
        out_type=jax.ShapeDtypeStruct((2, NP, D), jnp.float32),
        mesh=plsc.VectorSubcoreMesh(core_axis_name="c", subcore_axis_name="s",
                                    num_cores=NC, num_subcores=NS),
        scratch_types=(
            [pltpu.VMEM((CHUNK,), jnp.int32)] * (2 * NCHAIN)
            + [pltpu.VMEM((CHUNK, D), jnp.float32)] * NCHAIN
            + [pltpu.VMEM_SHARED((NP, D), jnp.float32)]
            + [pltpu.SemaphoreType.DMA] * NCHAIN
        ),
    )(x, src, dst)


BLK = 2000
NBLK = N // BLK


def _mlp1_body(x_ref, a0_ref, a1_ref, wa_ref, ba_ref, wb_ref, bb_ref, o_ref):
    h = x_ref[...] + a0_ref[0] + a1_ref[0]
    t = jnp.dot(h, wa_ref[...], preferred_element_type=jnp.float32) + ba_ref[...]
    t = jnp.maximum(t, 0.0)
    o = jnp.dot(t, wb_ref[...], preferred_element_type=jnp.float32) + bb_ref[...]
    o_ref[...] = jnp.maximum(o, 0.0)


def _mlp1(x, p, wa, ba, wb, bb):
    return pl.pallas_call(
        _mlp1_body,
        grid=(NBLK,),
        in_specs=[
            pl.BlockSpec((BLK, D), lambda i: (i, 0)),
            pl.BlockSpec((1, BLK, D), lambda i: (0, i, 0)),
            pl.BlockSpec((1, BLK, D), lambda i: (1, i, 0)),
            pl.BlockSpec((D, D), lambda i: (0, 0)),
            pl.BlockSpec((1, D), lambda i: (0, 0)),
            pl.BlockSpec((D, D), lambda i: (0, 0)),
            pl.BlockSpec((1, D), lambda i: (0, 0)),
        ],
        out_specs=pl.BlockSpec((BLK, D), lambda i: (i, 0)),
        out_shape=jax.ShapeDtypeStruct((N, D), jnp.float32),
    )(x, p, p, wa, ba, wb, bb)


def _mlp2_pool_body(h1_ref, a0_ref, a1_ref, wa_ref, ba_ref, wb_ref, bb_ref,
                    batch_ref, wfc_ref, bfc_ref, o_ref, acc_ref):
    i = pl.program_id(0)
    h = h1_ref[...] + a0_ref[0] + a1_ref[0]
    t = jnp.dot(h, wa_ref[...], preferred_element_type=jnp.float32) + ba_ref[...]
    t = jnp.maximum(t, 0.0)
    h2 = jnp.dot(t, wb_ref[...], preferred_element_type=jnp.float32) + bb_ref[...]
    h2 = jnp.maximum(h2, 0.0)

    onehot = (batch_ref[...] ==
              lax.broadcasted_iota(jnp.int32, (BLK, G), 1)).astype(jnp.float32)
    pooled = lax.dot_general(onehot, h2, (((0,), (0,)), ((), ())),
                             preferred_element_type=jnp.float32)

    @pl.when(i == 0)
    def _():
        acc_ref[...] = jnp.zeros_like(acc_ref)

    acc_ref[...] += pooled

    @pl.when(i == NBLK - 1)
    def _():
        logits = (jnp.dot(acc_ref[...], wfc_ref[...],
                          preferred_element_type=jnp.float32) + bfc_ref[...])
        m = jnp.max(logits, axis=1, keepdims=True)
        lse = jnp.log(jnp.sum(jnp.exp(logits - m), axis=1, keepdims=True)) + m
        o_ref[...] = logits - lse


def _mlp2_pool(h1, p, wa, ba, wb, bb, batch2d, wfc, bfc):
    return pl.pallas_call(
        _mlp2_pool_body,
        grid=(NBLK,),
        in_specs=[
            pl.BlockSpec((BLK, D), lambda i: (i, 0)),
            pl.BlockSpec((1, BLK, D), lambda i: (0, i, 0)),
            pl.BlockSpec((1, BLK, D), lambda i: (1, i, 0)),
            pl.BlockSpec((D, D), lambda i: (0, 0)),
            pl.BlockSpec((1, D), lambda i: (0, 0)),
            pl.BlockSpec((D, D), lambda i: (0, 0)),
            pl.BlockSpec((1, D), lambda i: (0, 0)),
            pl.BlockSpec((BLK, 1), lambda i: (i, 0)),
            pl.BlockSpec((D, D), lambda i: (0, 0)),
            pl.BlockSpec((1, D), lambda i: (0, 0)),
        ],
        out_specs=pl.BlockSpec((G, D), lambda i: (0, 0)),
        out_shape=jax.ShapeDtypeStruct((G, D), jnp.float32),
        scratch_shapes=[pltpu.VMEM((G, D), jnp.float32)],
    )(h1, p, p, wa, ba, wb, bb, batch2d, wfc, bfc)


def kernel(x, edge_index, batch, W1a, b1a, W1b, b1b, W2a, b2a, W2b, b2b,
           Wfc, bfc):
    src = edge_index[0].astype(jnp.int32)
    dst = edge_index[1].astype(jnp.int32)
    pad = E2 - E
    padi = jnp.arange(pad, dtype=jnp.int32)
    srcp = jnp.concatenate([src, padi % N])
    dstp = jnp.concatenate([dst, N + padi % (NP - N)])
    batch2d = batch.astype(jnp.int32).reshape(N, 1)

    p1 = _sc_agg(x, srcp, dstp)
    h1 = _mlp1(x, p1, W1a, b1a.reshape(1, D), W1b, b1b.reshape(1, D))
    p2 = _sc_agg(h1, srcp, dstp)
    return _mlp2_pool(h1, p2, W2a, b2a.reshape(1, D), W2b, b2b.reshape(1, D),
                      batch2d, Wfc, bfc.reshape(1, D))
